# Initial kernel scaffold; baseline (speedup 1.0000x reference)
import jax
import jax.numpy as jnp
from jax.experimental import pallas as pl


def kernel(author_x, paper_x, src_index, dst_index, W, b):
    raise NotImplementedError("write your pallas kernel here")


# speedup vs baseline: 1.0363x; 1.0363x over previous
import functools
import jax
import jax.numpy as jnp
from jax import lax
from jax.experimental import pallas as pl
from jax.experimental.pallas import tpu as pltpu
from jax.experimental.pallas import tpu_sc as plsc

B = 16384
NC = 2
NS = 16
L = 16
NW = NC * NS
BPW = B // NW


@functools.partial(
    pl.kernel,
    out_type=jax.ShapeDtypeStruct((B,), jnp.float32),
    mesh=plsc.VectorSubcoreMesh(core_axis_name="c", subcore_axis_name="s"),
    scratch_types=[
        pltpu.VMEM((BPW,), jnp.int32),
        pltpu.VMEM((BPW,), jnp.int32),
        pltpu.VMEM((BPW,), jnp.float32),
        pltpu.VMEM((BPW,), jnp.float32),
        pltpu.VMEM((BPW,), jnp.float32),
        pltpu.VMEM((3, L), jnp.float32),
        pltpu.SemaphoreType.DMA,
    ],
)
def _sc_link_scores(author_hbm, paper_hbm, src_hbm, dst_hbm, params_hbm,
                    out_hbm, sidx_v, didx_v, srow_v, drow_v, out_v, par_v,
                    sem):
    wid = lax.axis_index("s") * NC + lax.axis_index("c")
    base = wid * BPW
    pltpu.sync_copy(src_hbm.at[pl.ds(base, BPW)], sidx_v)
    pltpu.sync_copy(dst_hbm.at[pl.ds(base, BPW)], didx_v)
    pltpu.sync_copy(params_hbm, par_v)
    cp_s = pltpu.async_copy(author_hbm.at[sidx_v], srow_v, sem)
    cp_d = pltpu.async_copy(paper_hbm.at[didx_v], drow_v, sem)
    cp_s.wait()
    cp_d.wait()
    w0 = par_v[0, :]
    w1 = par_v[1, :]
    bb = par_v[2, :]
    for i in range(BPW // L):
        sl = pl.ds(i * L, L)
        out_v[sl] = srow_v[sl] * w0 + drow_v[sl] * w1 + bb
    pltpu.sync_copy(out_v, out_hbm.at[pl.ds(base, BPW)])


def kernel(author_x, paper_x, src_index, dst_index, W, b):
    params = jnp.stack([
        jnp.broadcast_to(W[0, 0], (L,)),
        jnp.broadcast_to(W[1, 0], (L,)),
        jnp.broadcast_to(b[0], (L,)),
    ])
    return _sc_link_scores(author_x[:, 0], paper_x[:, 0],
                           src_index, dst_index, params)
